# baseline (speedup 1.0000x reference)
import jax
import jax.numpy as jnp
from jax.experimental import pallas as pl
from jax.experimental.pallas import tpu as pltpu

N = 8192
F = 1024
E = 64
T = 4
SCALE = F ** (-0.5)
TILE1 = 1024
TILE3 = 1024
PREC = jax.lax.Precision.DEFAULT
_DNT = (((1,), (1,)), ((), ()))
_DN0 = (((0,), (0,)), ((), ()))


def _onehot_argmax(lg):
    m = jnp.max(lg, axis=1, keepdims=True)
    io = jax.lax.broadcasted_iota(jnp.int32, lg.shape, 1)
    idx = jnp.min(jnp.where(lg == m, io, E), axis=1, keepdims=True)
    return (io == idx).astype(jnp.float32)


def _phase1(f_ref, w0_ref, w1_ref, w2_ref, w3_ref, bt_ref, wb0_ref, bb0_ref,
            wb1_ref, bb1_ref, we_ref, be_ref, hm_ref, hf_ref, cs_ref,
            z0b_ref, mb_ref, hfa_ref, csa_ref):
    f = f_ref[...]
    wb0 = wb0_ref[...]
    bb0 = bb0_ref[...]
    wb1 = wb1_ref[...]
    bb1 = bb1_ref[...]
    we = we_ref[...]
    be = be_ref[...]

    def tail(af):
        h = jax.lax.dot_general(
            jnp.maximum(af, 0.0), wb0, _DNT, precision=PREC) + bb0
        z = jax.lax.dot_general(
            jnp.maximum(h, 0.0), wb1, _DNT, precision=PREC) + bb1
        lg = jax.lax.dot_general(
            jnp.maximum(z, 0.0), we, _DNT, precision=PREC) + be
        return z, lg

    @pl.when(pl.program_id(0) != 0)
    def _acc():
        hfa_ref[...] += jax.lax.dot_general(
            mb_ref[...], z0b_ref[...], _DN0,
            preferred_element_type=jnp.float32)

    hm = jnp.zeros((TILE1, E), jnp.float32)
    for i, w_ref in enumerate((w0_ref, w1_ref, w2_ref, w3_ref)):
        af = jax.lax.dot_general(
            f, w_ref[...], _DNT, precision=PREC) + bt_ref[i]
        _, lg = tail(af)
        hm = hm + _onehot_argmax(lg)
    z0, lg0 = tail(f)
    hm = hm + _onehot_argmax(lg0)

    hm_ref[...] = hm
    mask = (hm > 0.0).astype(jnp.bfloat16)
    csum = jnp.sum(jnp.exp(hm), axis=0, keepdims=True)

    @pl.when(pl.program_id(0) == 0)
    def _init():
        hfa_ref[...] = jnp.zeros((E, F), jnp.float32)
        csa_ref[...] = csum

    @pl.when(pl.program_id(0) != 0)
    def _acc2():
        csa_ref[...] += csum

    z0b_ref[...] = z0.astype(jnp.bfloat16)
    mb_ref[...] = mask
    hf_ref[...] = hfa_ref[...]
    cs_ref[...] = csa_ref[...]


def _phase2(f_ref, hfi_ref, csi_ref, z0b_ref, mb_ref,
            w0_ref, w1_ref, w2_ref, w3_ref, bt_ref,
            hm_ref, hf_ref, dots_ref, hs_ref, g_ref, c_ref, hffull_ref):
    @pl.when(pl.program_id(0) == 0)
    def _prep():
        hf = hfi_ref[...] + jax.lax.dot_general(
            mb_ref[...], z0b_ref[...], _DN0,
            preferred_element_type=jnp.float32)
        hffull_ref[...] = hf
        hfb = hf.astype(jnp.bfloat16)
        g_ref[0:E] = hfb
        c_ref[:, 0:E] = jnp.zeros((1, E), jnp.float32)
        for i, w_ref in enumerate((w0_ref, w1_ref, w2_ref, w3_ref)):
            g_ref[(i + 1) * E:(i + 2) * E] = jnp.dot(
                hf, w_ref[...], precision=PREC).astype(jnp.bfloat16)
            c_ref[:, (i + 1) * E:(i + 2) * E] = jax.lax.dot_general(
                bt_ref[i], hf, _DNT, precision=PREC)

    hf_ref[...] = hffull_ref[...]
    hs_ref[...] = jnp.exp(hm_ref[...]) / csi_ref[...]
    fb = f_ref[...].astype(jnp.bfloat16)
    c = c_ref[...]
    d01 = (jax.lax.dot_general(fb, g_ref[0:2 * E], _DNT,
                               preferred_element_type=jnp.float32)
           + c[:, 0:2 * E]) * SCALE
    dots_ref[0] = d01[:, :E]
    dots_ref[1] = d01[:, E:]
    d23 = (jax.lax.dot_general(fb, g_ref[2 * E:4 * E], _DNT,
                               preferred_element_type=jnp.float32)
           + c[:, 2 * E:4 * E]) * SCALE
    dots_ref[2] = d23[:, :E]
    dots_ref[3] = d23[:, E:]
    d4 = (jax.lax.dot_general(fb, g_ref[4 * E:5 * E], _DNT,
                              preferred_element_type=jnp.float32)
          + c[:, 4 * E:5 * E]) * SCALE
    dots_ref[4] = d4


def kernel(features, W0, b0, W1, b1, W2, b2, W3, b3, Wb0, bb0, Wb1, bb1, We, be):
    bst = jnp.stack([b0, b1, b2, b3])[:, None, :]
    bb0r = bb0[None, :]
    bb1r = bb1[None, :]
    ber = be[None, :]

    wspec = pl.BlockSpec((F, F), lambda i: (0, 0))
    rt1 = N // TILE1
    hm, hfo, cso, z0b, mb = pl.pallas_call(
        _phase1,
        grid=(rt1,),
        in_specs=[
            pl.BlockSpec((TILE1, F), lambda i: (i, 0)),
            wspec, wspec, wspec, wspec,
            pl.BlockSpec((T, 1, F), lambda i: (0, 0, 0)),
            wspec,
            pl.BlockSpec((1, F), lambda i: (0, 0)),
            wspec,
            pl.BlockSpec((1, F), lambda i: (0, 0)),
            pl.BlockSpec((E, F), lambda i: (0, 0)),
            pl.BlockSpec((1, E), lambda i: (0, 0)),
        ],
        out_specs=[
            pl.BlockSpec((TILE1, E), lambda i: (i, 0)),
            pl.BlockSpec((E, F), lambda i: (0, 0)),
            pl.BlockSpec((1, E), lambda i: (0, 0)),
            pl.BlockSpec((TILE1, F), lambda i: (0, 0)),
            pl.BlockSpec((TILE1, E), lambda i: (0, 0)),
        ],
        out_shape=[
            jax.ShapeDtypeStruct((N, E), jnp.float32),
            jax.ShapeDtypeStruct((E, F), jnp.float32),
            jax.ShapeDtypeStruct((1, E), jnp.float32),
            jax.ShapeDtypeStruct((TILE1, F), jnp.bfloat16),
            jax.ShapeDtypeStruct((TILE1, E), jnp.bfloat16),
        ],
        scratch_shapes=[
            pltpu.VMEM((E, F), jnp.float32),
            pltpu.VMEM((1, E), jnp.float32),
        ],
        compiler_params=pltpu.CompilerParams(
            dimension_semantics=("arbitrary",)),
    )(features, W0, W1, W2, W3, bst, Wb0, bb0r, Wb1, bb1r, We, ber)

    rt3 = N // TILE3
    hf, dots5, hs = pl.pallas_call(
        _phase2,
        grid=(rt3,),
        in_specs=[
            pl.BlockSpec((TILE3, F), lambda i: (i, 0)),
            pl.BlockSpec((E, F), lambda i: (0, 0)),
            pl.BlockSpec((1, E), lambda i: (0, 0)),
            pl.BlockSpec((TILE1, F), lambda i: (0, 0)),
            pl.BlockSpec((TILE1, E), lambda i: (0, 0)),
            wspec, wspec, wspec, wspec,
            pl.BlockSpec((T, 1, F), lambda i: (0, 0, 0)),
            pl.BlockSpec((TILE3, E), lambda i: (i, 0)),
        ],
        out_specs=[
            pl.BlockSpec((E, F), lambda i: (0, 0)),
            pl.BlockSpec((T + 1, TILE3, E), lambda i: (0, i, 0)),
            pl.BlockSpec((TILE3, E), lambda i: (i, 0)),
        ],
        out_shape=[
            jax.ShapeDtypeStruct((E, F), jnp.float32),
            jax.ShapeDtypeStruct((T + 1, N, E), jnp.float32),
            jax.ShapeDtypeStruct((N, E), jnp.float32),
        ],
        scratch_shapes=[
            pltpu.VMEM(((T + 1) * E, F), jnp.bfloat16),
            pltpu.VMEM((1, (T + 1) * E), jnp.float32),
            pltpu.VMEM((E, F), jnp.float32),
        ],
        compiler_params=pltpu.CompilerParams(
            dimension_semantics=("arbitrary",)),
    )(features, hfo, cso, z0b, mb, W0, W1, W2, W3, bst, hm)

    dots = dots5.reshape((T + 1) * N, E)
    return (hs, hf, dots)


# speedup vs baseline: 1.0857x; 1.0857x over previous
import jax
import jax.numpy as jnp
from jax.experimental import pallas as pl
from jax.experimental.pallas import tpu as pltpu

N = 8192
F = 1024
E = 64
T = 4
SCALE = F ** (-0.5)
TILE = 1024
RT = N // TILE
PREC = jax.lax.Precision.DEFAULT
_DNT = (((1,), (1,)), ((), ()))
_DN0 = (((0,), (0,)), ((), ()))


def _onehot_argmax(lg):
    m = jnp.max(lg, axis=1, keepdims=True)
    io = jax.lax.broadcasted_iota(jnp.int32, lg.shape, 1)
    idx = jnp.min(jnp.where(lg == m, io, E), axis=1, keepdims=True)
    return (io == idx).astype(jnp.float32)


def _body(f_ref, w0_ref, w1_ref, w2_ref, w3_ref, bt_ref, wb0_ref, bb0_ref,
          wb1_ref, bb1_ref, we_ref, be_ref, hf_ref, dots_ref, hs_ref,
          hfa_ref, csa_ref, hms_ref, g_ref, c_ref, ds_ref, hss_ref):
    i = pl.program_id(0)

    @pl.when(i < RT)
    def _p1():
        f = f_ref[...]
        wb0 = wb0_ref[...]
        bb0 = bb0_ref[...]
        wb1 = wb1_ref[...]
        bb1 = bb1_ref[...]
        we = we_ref[...]
        be = be_ref[...]

        def tail(af):
            h = jax.lax.dot_general(
                jnp.maximum(af, 0.0), wb0, _DNT, precision=PREC) + bb0
            z = jax.lax.dot_general(
                jnp.maximum(h, 0.0), wb1, _DNT, precision=PREC) + bb1
            lg = jax.lax.dot_general(
                jnp.maximum(z, 0.0), we, _DNT, precision=PREC) + be
            return z, lg

        hm = jnp.zeros((TILE, E), jnp.float32)
        for k, w_ref in enumerate((w0_ref, w1_ref, w2_ref, w3_ref)):
            af = jax.lax.dot_general(
                f, w_ref[...], _DNT, precision=PREC) + bt_ref[k]
            _, lg = tail(af)
            hm = hm + _onehot_argmax(lg)
        z0, lg0 = tail(f)
        hm = hm + _onehot_argmax(lg0)

        hms_ref[pl.ds(i * TILE, TILE), :] = hm
        mask = (hm > 0.0).astype(jnp.bfloat16)
        part = jax.lax.dot_general(
            mask, z0.astype(jnp.bfloat16), _DN0,
            preferred_element_type=jnp.float32)
        csum = jnp.sum(jnp.exp(hm), axis=0, keepdims=True)

        @pl.when(i == 0)
        def _init():
            hfa_ref[...] = part
            csa_ref[...] = csum

        @pl.when(i != 0)
        def _acc():
            hfa_ref[...] += part
            csa_ref[...] += csum

    @pl.when(i == RT)
    def _prep():
        hf = hfa_ref[...]
        g_ref[0:E] = hf.astype(jnp.bfloat16)
        c_ref[:, 0:E] = jnp.zeros((1, E), jnp.float32)
        for k, w_ref in enumerate((w0_ref, w1_ref, w2_ref, w3_ref)):
            g_ref[(k + 1) * E:(k + 2) * E] = jnp.dot(
                hf, w_ref[...], precision=PREC).astype(jnp.bfloat16)
            c_ref[:, (k + 1) * E:(k + 2) * E] = jax.lax.dot_general(
                bt_ref[k], hf, _DNT, precision=PREC)

    @pl.when(i >= RT)
    def _p3():
        j = i - RT
        hm = hms_ref[pl.ds(j * TILE, TILE), :]
        hss_ref[...] = jnp.exp(hm) / csa_ref[...]
        fb = f_ref[...].astype(jnp.bfloat16)
        c = c_ref[...]
        d01 = (jax.lax.dot_general(fb, g_ref[0:2 * E], _DNT,
                                   preferred_element_type=jnp.float32)
               + c[:, 0:2 * E]) * SCALE
        ds_ref[0] = d01[:, :E]
        ds_ref[1] = d01[:, E:]
        d23 = (jax.lax.dot_general(fb, g_ref[2 * E:4 * E], _DNT,
                                   preferred_element_type=jnp.float32)
               + c[:, 2 * E:4 * E]) * SCALE
        ds_ref[2] = d23[:, :E]
        ds_ref[3] = d23[:, E:]
        ds_ref[4] = (jax.lax.dot_general(fb, g_ref[4 * E:5 * E], _DNT,
                                         preferred_element_type=jnp.float32)
                     + c[:, 4 * E:5 * E]) * SCALE

    hf_ref[...] = hfa_ref[...]
    dots_ref[...] = ds_ref[...]
    hs_ref[...] = hss_ref[...]


def kernel(features, W0, b0, W1, b1, W2, b2, W3, b3, Wb0, bb0, Wb1, bb1, We, be):
    bst = jnp.stack([b0, b1, b2, b3])[:, None, :]
    bb0r = bb0[None, :]
    bb1r = bb1[None, :]
    ber = be[None, :]

    wspec = pl.BlockSpec((F, F), lambda i: (0, 0))

    def fmap(i):
        return (jnp.where(i < RT, i, i - RT), 0)

    def omap(i):
        return (jnp.maximum(i - RT, 0), 0)

    hf, dots5, hs = pl.pallas_call(
        _body,
        grid=(2 * RT,),
        in_specs=[
            pl.BlockSpec((TILE, F), fmap),
            wspec, wspec, wspec, wspec,
            pl.BlockSpec((T, 1, F), lambda i: (0, 0, 0)),
            wspec,
            pl.BlockSpec((1, F), lambda i: (0, 0)),
            wspec,
            pl.BlockSpec((1, F), lambda i: (0, 0)),
            pl.BlockSpec((E, F), lambda i: (0, 0)),
            pl.BlockSpec((1, E), lambda i: (0, 0)),
        ],
        out_specs=[
            pl.BlockSpec((E, F), lambda i: (0, 0)),
            pl.BlockSpec((T + 1, TILE, E),
                         lambda i: (0, jnp.maximum(i - RT, 0), 0)),
            pl.BlockSpec((TILE, E), omap),
        ],
        out_shape=[
            jax.ShapeDtypeStruct((E, F), jnp.float32),
            jax.ShapeDtypeStruct((T + 1, N, E), jnp.float32),
            jax.ShapeDtypeStruct((N, E), jnp.float32),
        ],
        scratch_shapes=[
            pltpu.VMEM((E, F), jnp.float32),
            pltpu.VMEM((1, E), jnp.float32),
            pltpu.VMEM((N, E), jnp.float32),
            pltpu.VMEM(((T + 1) * E, F), jnp.bfloat16),
            pltpu.VMEM((1, (T + 1) * E), jnp.float32),
            pltpu.VMEM((T + 1, TILE, E), jnp.float32),
            pltpu.VMEM((TILE, E), jnp.float32),
        ],
        compiler_params=pltpu.CompilerParams(
            dimension_semantics=("arbitrary",)),
    )(features, W0, W1, W2, W3, bst, Wb0, bb0r, Wb1, bb1r, We, ber)

    dots = dots5.reshape((T + 1) * N, E)
    return (hs, hf, dots)


# speedup vs baseline: 1.3616x; 1.2542x over previous
import jax
import jax.numpy as jnp
from jax.experimental import pallas as pl
from jax.experimental.pallas import tpu as pltpu

N = 8192
F = 1024
E = 64
T = 4
SCALE = F ** (-0.5)
TILE1 = 1024
TILE3 = 1024
RT3 = N // TILE3
PREC = jax.lax.Precision.DEFAULT
_DNT = (((1,), (1,)), ((), ()))
_DN0 = (((0,), (0,)), ((), ()))


def _onehot_argmax(lg):
    m = jnp.max(lg, axis=1, keepdims=True)
    io = jax.lax.broadcasted_iota(jnp.int32, lg.shape, 1)
    idx = jnp.min(jnp.where(lg == m, io, E), axis=1, keepdims=True)
    return (io == idx).astype(jnp.float32)


def _phase1(f_ref, w0_ref, w1_ref, w2_ref, w3_ref,
            b0_ref, b1_ref, b2_ref, b3_ref, wb0_ref, bb0_ref,
            wb1_ref, bb1_ref, we_ref, be_ref, hm_ref, hf_ref, cs_ref,
            hfa_ref, csa_ref):
    f = f_ref[...]
    wb0 = wb0_ref[...]
    bb0 = bb0_ref[...]
    wb1 = wb1_ref[...]
    bb1 = bb1_ref[...]
    we = we_ref[...]
    be = be_ref[...]

    def tail(af):
        h = jax.lax.dot_general(
            jnp.maximum(af, 0.0), wb0, _DNT, precision=PREC) + bb0
        z = jax.lax.dot_general(
            jnp.maximum(h, 0.0), wb1, _DNT, precision=PREC) + bb1
        lg = jax.lax.dot_general(
            jnp.maximum(z, 0.0), we, _DNT, precision=PREC) + be
        return z, lg

    hm = jnp.zeros((TILE1, E), jnp.float32)
    for w_ref, b_ref in ((w0_ref, b0_ref), (w1_ref, b1_ref),
                         (w2_ref, b2_ref), (w3_ref, b3_ref)):
        af = jax.lax.dot_general(
            f, w_ref[...], _DNT, precision=PREC) + b_ref[...]
        _, lg = tail(af)
        hm = hm + _onehot_argmax(lg)
    z0, lg0 = tail(f)
    hm = hm + _onehot_argmax(lg0)

    hm_ref[...] = hm
    mask = (hm > 0.0).astype(jnp.bfloat16)
    part = jax.lax.dot_general(
        mask, z0.astype(jnp.bfloat16), _DN0,
        preferred_element_type=jnp.float32)
    csum = jnp.sum(jnp.exp(hm), axis=0, keepdims=True)

    @pl.when(pl.program_id(0) == 0)
    def _init():
        hfa_ref[...] = part
        csa_ref[...] = csum

    @pl.when(pl.program_id(0) != 0)
    def _acc():
        hfa_ref[...] += part
        csa_ref[...] += csum

    hf_ref[...] = hfa_ref[...]
    cs_ref[...] = csa_ref[...]


def _phase2(f_ref, hfi_ref, csi_ref, w0_ref, w1_ref, w2_ref, w3_ref,
            b0_ref, b1_ref, b2_ref, b3_ref,
            hm_ref, hf_ref, dots_ref, hs_ref, g_ref, c_ref):
    i = pl.program_id(0)
    b = pl.program_id(1)

    @pl.when((i == 0) & (b == 0))
    def _prep():
        hf = hfi_ref[...]
        g_ref[0] = hf.astype(jnp.bfloat16)
        c_ref[0] = jnp.zeros((1, E), jnp.float32)
        for k, (w_ref, b_ref) in enumerate(
                ((w0_ref, b0_ref), (w1_ref, b1_ref),
                 (w2_ref, b2_ref), (w3_ref, b3_ref))):
            g_ref[k + 1] = jnp.dot(
                hf, w_ref[...], precision=PREC).astype(jnp.bfloat16)
            c_ref[k + 1] = jax.lax.dot_general(
                b_ref[...], hf, _DNT, precision=PREC)

    hf_ref[...] = hfi_ref[...]
    hs_ref[...] = jnp.exp(hm_ref[...]) / csi_ref[...]
    fb = f_ref[...].astype(jnp.bfloat16)
    gb = g_ref[b]
    cb = c_ref[b]
    d = jax.lax.dot_general(fb, gb, _DNT, preferred_element_type=jnp.float32)
    dots_ref[...] = (d + cb) * SCALE


def kernel(features, W0, b0, W1, b1, W2, b2, W3, b3, Wb0, bb0, Wb1, bb1, We, be):
    b0r = b0[None, :]
    b1r = b1[None, :]
    b2r = b2[None, :]
    b3r = b3[None, :]
    bb0r = bb0[None, :]
    bb1r = bb1[None, :]
    ber = be[None, :]

    wspec = pl.BlockSpec((F, F), lambda i: (0, 0))
    bspec = pl.BlockSpec((1, F), lambda i: (0, 0))
    rt1 = N // TILE1
    hm, hfo, cso = pl.pallas_call(
        _phase1,
        grid=(rt1,),
        in_specs=[
            pl.BlockSpec((TILE1, F), lambda i: (i, 0)),
            wspec, wspec, wspec, wspec,
            bspec, bspec, bspec, bspec,
            wspec,
            bspec,
            wspec,
            bspec,
            pl.BlockSpec((E, F), lambda i: (0, 0)),
            pl.BlockSpec((1, E), lambda i: (0, 0)),
        ],
        out_specs=[
            pl.BlockSpec((TILE1, E), lambda i: (i, 0)),
            pl.BlockSpec((E, F), lambda i: (0, 0)),
            pl.BlockSpec((1, E), lambda i: (0, 0)),
        ],
        out_shape=[
            jax.ShapeDtypeStruct((N, E), jnp.float32),
            jax.ShapeDtypeStruct((E, F), jnp.float32),
            jax.ShapeDtypeStruct((1, E), jnp.float32),
        ],
        scratch_shapes=[
            pltpu.VMEM((E, F), jnp.float32),
            pltpu.VMEM((1, E), jnp.float32),
        ],
        compiler_params=pltpu.CompilerParams(
            dimension_semantics=("arbitrary",)),
    )(features, W0, W1, W2, W3, b0r, b1r, b2r, b3r,
      Wb0, bb0r, Wb1, bb1r, We, ber)

    wspec2 = pl.BlockSpec((F, F), lambda i, b: (0, 0))
    bspec2 = pl.BlockSpec((1, F), lambda i, b: (0, 0))
    hf, dots, hs = pl.pallas_call(
        _phase2,
        grid=(RT3, T + 1),
        in_specs=[
            pl.BlockSpec((TILE3, F), lambda i, b: (i, 0)),
            pl.BlockSpec((E, F), lambda i, b: (0, 0)),
            pl.BlockSpec((1, E), lambda i, b: (0, 0)),
            wspec2, wspec2, wspec2, wspec2,
            bspec2, bspec2, bspec2, bspec2,
            pl.BlockSpec((TILE3, E), lambda i, b: (i, 0)),
        ],
        out_specs=[
            pl.BlockSpec((E, F), lambda i, b: (0, 0)),
            pl.BlockSpec((TILE3, E), lambda i, b: (b * RT3 + i, 0)),
            pl.BlockSpec((TILE3, E), lambda i, b: (i, 0)),
        ],
        out_shape=[
            jax.ShapeDtypeStruct((E, F), jnp.float32),
            jax.ShapeDtypeStruct(((T + 1) * N, E), jnp.float32),
            jax.ShapeDtypeStruct((N, E), jnp.float32),
        ],
        scratch_shapes=[
            pltpu.VMEM((T + 1, E, F), jnp.bfloat16),
            pltpu.VMEM((T + 1, 1, E), jnp.float32),
        ],
        compiler_params=pltpu.CompilerParams(
            dimension_semantics=("arbitrary", "arbitrary")),
    )(features, hfo, cso, W0, W1, W2, W3, b0r, b1r, b2r, b3r, hm)

    return (hs, hf, dots)
